# baseline (speedup 1.0000x reference)
import jax
import jax.numpy as jnp
from jax.experimental import pallas as pl
from jax.experimental.pallas import tpu as pltpu

_N = 33554432
_COLS = 1024
_ROWS = _N // _COLS
_FIX_ROWS = 8
_NCHUNKS = 16

_body_rows = _ROWS - _FIX_ROWS
_chunk = -(-_body_rows // _NCHUNKS)
_starts_sizes = []
_s = _FIX_ROWS
while _s < _ROWS:
    _n = min(_chunk, _ROWS - _s)
    _starts_sizes.append((_s, _n))
    _s += _n


def _copy_body(x_ref, o_ref, buf, fix_sem, sem):
    fix_in = pltpu.make_async_copy(x_ref.at[pl.ds(0, _FIX_ROWS)], buf, fix_sem)
    fix_in.start()
    copies = [
        pltpu.make_async_copy(
            x_ref.at[pl.ds(s, n)], o_ref.at[pl.ds(s, n)], sem
        )
        for s, n in _starts_sizes
    ]
    for c in copies:
        c.start()
    fix_in.wait()
    col = jax.lax.broadcasted_iota(jnp.int32, (1, _COLS), 1)
    buf[0:1, :] = jnp.where(col == 0, 0.0, buf[0:1, :])
    fix_out = pltpu.make_async_copy(buf, o_ref.at[pl.ds(0, _FIX_ROWS)], fix_sem)
    fix_out.start()
    fix_out.wait()
    for c in copies:
        c.wait()


def kernel(x):
    x2 = x.reshape(_ROWS, _COLS)
    out = pl.pallas_call(
        _copy_body,
        out_shape=jax.ShapeDtypeStruct((_ROWS, _COLS), x.dtype),
        in_specs=[pl.BlockSpec(memory_space=pltpu.MemorySpace.HBM)],
        out_specs=pl.BlockSpec(memory_space=pltpu.MemorySpace.HBM),
        scratch_shapes=[
            pltpu.VMEM((_FIX_ROWS, _COLS), jnp.float32),
            pltpu.SemaphoreType.DMA,
            pltpu.SemaphoreType.DMA,
        ],
    )(x2)
    return out.reshape(_N)


# speedup vs baseline: 13.2845x; 13.2845x over previous
import jax
import jax.numpy as jnp
from jax.experimental import pallas as pl
from jax.experimental.pallas import tpu as pltpu

_N = 33554432
_COLS = 1024
_ROWS = _N // _COLS
_CH_ROWS = 512
_NCHUNKS = _ROWS // _CH_ROWS
_D = 4
_NBUF = 2 * _D


def _copy_body(x_ref, o_ref, buf, sem_in, sem_out):
    def in_cp(i):
        return pltpu.make_async_copy(
            x_ref.at[pl.ds(i * _CH_ROWS, _CH_ROWS)],
            buf.at[i % _NBUF],
            sem_in.at[i % _NBUF],
        )

    def out_cp(i):
        return pltpu.make_async_copy(
            buf.at[i % _NBUF],
            o_ref.at[pl.ds(i * _CH_ROWS, _CH_ROWS)],
            sem_out.at[i % _NBUF],
        )

    for i in range(_D):
        in_cp(i).start()
    for i in range(_NCHUNKS):
        in_cp(i).wait()
        if i == 0:
            col = jax.lax.broadcasted_iota(jnp.int32, (1, _COLS), 1)
            buf[0, 0:1, :] = jnp.where(col == 0, 0.0, buf[0, 0:1, :])
        out_cp(i).start()
        j = i + _D
        if j < _NCHUNKS:
            if j >= _NBUF:
                out_cp(j - _NBUF).wait()
            in_cp(j).start()
    for i in range(_NCHUNKS - _NBUF, _NCHUNKS):
        out_cp(i).wait()


def kernel(x):
    x2 = x.reshape(_ROWS, _COLS)
    out = pl.pallas_call(
        _copy_body,
        out_shape=jax.ShapeDtypeStruct((_ROWS, _COLS), x.dtype),
        in_specs=[pl.BlockSpec(memory_space=pltpu.MemorySpace.HBM)],
        out_specs=pl.BlockSpec(memory_space=pltpu.MemorySpace.HBM),
        scratch_shapes=[
            pltpu.VMEM((_NBUF, _CH_ROWS, _COLS), jnp.float32),
            pltpu.SemaphoreType.DMA((_NBUF,)),
            pltpu.SemaphoreType.DMA((_NBUF,)),
        ],
    )(x2)
    return out.reshape(_N)


# speedup vs baseline: 51.8159x; 3.9005x over previous
import jax
import jax.numpy as jnp
from jax.experimental import pallas as pl
from jax.experimental.pallas import tpu as pltpu

_N = 33554432
_CH = 1 << 19
_NCHUNKS = _N // _CH
_D = 4
_NBUF = 2 * _D


def _copy_body(x_ref, o_ref, buf, sem_in, sem_out):
    def in_cp(i):
        return pltpu.make_async_copy(
            x_ref.at[pl.ds(i * _CH, _CH)],
            buf.at[i % _NBUF],
            sem_in.at[i % _NBUF],
        )

    def out_cp(i):
        return pltpu.make_async_copy(
            buf.at[i % _NBUF],
            o_ref.at[pl.ds(i * _CH, _CH)],
            sem_out.at[i % _NBUF],
        )

    for i in range(_D):
        in_cp(i).start()
    for i in range(_NCHUNKS):
        in_cp(i).wait()
        if i == 0:
            idx = jax.lax.broadcasted_iota(jnp.int32, (128,), 0)
            buf[0, 0:128] = jnp.where(idx == 0, 0.0, buf[0, 0:128])
        out_cp(i).start()
        j = i + _D
        if j < _NCHUNKS:
            if j >= _NBUF:
                out_cp(j - _NBUF).wait()
            in_cp(j).start()
    for i in range(_NCHUNKS - _NBUF, _NCHUNKS):
        out_cp(i).wait()


def kernel(x):
    return pl.pallas_call(
        _copy_body,
        out_shape=jax.ShapeDtypeStruct((_N,), x.dtype),
        in_specs=[pl.BlockSpec(memory_space=pltpu.MemorySpace.HBM)],
        out_specs=pl.BlockSpec(memory_space=pltpu.MemorySpace.HBM),
        scratch_shapes=[
            pltpu.VMEM((_NBUF, _CH), jnp.float32),
            pltpu.SemaphoreType.DMA((_NBUF,)),
            pltpu.SemaphoreType.DMA((_NBUF,)),
        ],
    )(x)
